# Initial kernel scaffold; baseline (speedup 1.0000x reference)
import jax
import jax.numpy as jnp
from jax.experimental import pallas as pl


def kernel(x):
    raise NotImplementedError("write your pallas kernel here")


# speedup vs baseline: 9.3203x; 9.3203x over previous
import jax
import jax.numpy as jnp
from jax import lax
from jax.experimental import pallas as pl
from jax.experimental.pallas import tpu as pltpu
from jax.experimental.pallas import tpu_sc as plsc

_TOPK = 8
_NB = 8
_B = 128
_E = 32768
_BS = _E // _NB
_NVEC = _BS // 16

_info = plsc.get_sparse_core_info()
_NC = _info.num_cores
_NS = _info.num_subcores
_NW = _NC * _NS
_UNITS = _B * _NB
_UPW = _UNITS // _NW


def _wta_body(x_hbm, out_hbm, in_v, out_v, mat_v):
    wid = lax.axis_index("s") * _NC + lax.axis_index("c")
    lane = lax.iota(jnp.int32, 16)
    neg = jnp.full((16,), -jnp.inf, jnp.float32)
    mat_v[_TOPK] = neg

    def unit_body(u, carry):
        unit = wid * _UPW + u
        row = unit // _NB
        col = (unit % _NB) * _BS
        pltpu.sync_copy(x_hbm.at[row, pl.ds(col, _BS)], in_v)

        def p1(i, r):
            v = in_v[pl.ds(i * 16, 16)]
            rl = list(r)
            for j in range(_TOPK):
                hi = jnp.maximum(rl[j], v)
                v = jnp.minimum(rl[j], v)
                rl[j] = hi
            return tuple(rl)

        r = lax.fori_loop(0, _NVEC, p1, (neg,) * _TOPK)
        for j in range(_TOPK):
            mat_v[j] = r[j]

        def ext(i, c):
            ptr, heads, _ = c
            m = jnp.max(heads)
            f = plsc.all_reduce_ffs(heads == m)
            ptr = ptr + (lane == f).astype(jnp.int32)
            heads = plsc.load_gather(mat_v, [ptr, lane])
            return ptr, heads, m

        z16 = jnp.zeros((16,), jnp.int32)
        _, _, t8 = lax.fori_loop(0, _TOPK, ext,
                                 (z16, r[0], jnp.zeros((), jnp.float32)))

        c_gt = jnp.zeros((), jnp.int32)
        for j in range(_TOPK):
            c_gt = c_gt + jnp.sum((r[j] > t8).astype(jnp.int32))
        need_eq = _TOPK - c_gt

        def p3(i, run):
            v = in_v[pl.ds(i * 16, 16)]
            eq = v == t8
            eqc = eq.astype(jnp.int32)
            cum = plsc.cumsum(eqc)
            keep = (v > t8) | (eq & ((cum + run) <= need_eq))
            out_v[pl.ds(i * 16, 16)] = jnp.where(keep, v, 0.0)
            return run + jnp.sum(eqc)

        lax.fori_loop(0, _NVEC, p3, jnp.zeros((), jnp.int32))
        pltpu.sync_copy(out_v, out_hbm.at[row, pl.ds(col, _BS)])
        return carry

    lax.fori_loop(0, _UPW, unit_body, 0)


@jax.jit
def kernel(x):
    mesh = plsc.VectorSubcoreMesh(core_axis_name="c", subcore_axis_name="s")
    f = pl.kernel(
        _wta_body,
        out_type=jax.ShapeDtypeStruct((_B, _E), jnp.float32),
        mesh=mesh,
        scratch_types=[
            pltpu.VMEM((_BS,), jnp.float32),
            pltpu.VMEM((_BS,), jnp.float32),
            pltpu.VMEM((_TOPK + 1, 16), jnp.float32),
        ],
        compiler_params=pltpu.CompilerParams(needs_layout_passes=False),
    )
    return f(x)


# speedup vs baseline: 10.0476x; 1.0780x over previous
import jax
import jax.numpy as jnp
from jax import lax
from jax.experimental import pallas as pl
from jax.experimental.pallas import tpu as pltpu
from jax.experimental.pallas import tpu_sc as plsc

_TOPK = 8
_NB = 8
_B = 128
_E = 32768
_BS = _E // _NB
_NVEC = _BS // 16

_info = plsc.get_sparse_core_info()
_NC = _info.num_cores
_NS = _info.num_subcores
_NW = _NC * _NS
_UNITS = _B * _NB
_UPW = _UNITS // _NW


def _wta_body(x_hbm, out_hbm, in_v, out_v, mat_v):
    wid = lax.axis_index("s") * _NC + lax.axis_index("c")
    lane = lax.iota(jnp.int32, 16)
    neg = jnp.full((16,), -jnp.inf, jnp.float32)
    mat_v[_TOPK] = neg

    def unit_body(u, carry):
        unit = wid * _UPW + u
        row = unit // _NB
        col = (unit % _NB) * _BS
        pltpu.sync_copy(x_hbm.at[row, pl.ds(col, _BS)], in_v)

        nch = 4
        span = _NVEC // nch

        def p1(i, r):
            rl = list(r)
            for c in range(nch):
                v = in_v[pl.ds((c * span + i) * 16, 16)]
                for j in range(_TOPK):
                    o = c * _TOPK + j
                    hi = jnp.maximum(rl[o], v)
                    v = jnp.minimum(rl[o], v)
                    rl[o] = hi
            return tuple(rl)

        rr = lax.fori_loop(0, span, p1, (neg,) * (_TOPK * nch))

        def merge2(a, b):
            c = [jnp.maximum(a[j], b[_TOPK - 1 - j]) for j in range(_TOPK)]
            for d in (4, 2, 1):
                for i in range(_TOPK):
                    if i & d:
                        continue
                    k = i | d
                    hi = jnp.maximum(c[i], c[k])
                    lo = jnp.minimum(c[i], c[k])
                    c[i], c[k] = hi, lo
            return c

        ch = [list(rr[c * _TOPK:(c + 1) * _TOPK]) for c in range(nch)]
        r = merge2(merge2(ch[0], ch[1]), merge2(ch[2], ch[3]))
        for j in range(_TOPK):
            mat_v[j] = r[j]

        def ext(i, c):
            ptr, heads, _ = c
            m = jnp.max(heads)
            f = plsc.all_reduce_ffs(heads == m)
            ptr = ptr + (lane == f).astype(jnp.int32)
            heads = plsc.load_gather(mat_v, [ptr, lane])
            return ptr, heads, m

        z16 = jnp.zeros((16,), jnp.int32)
        _, _, t8 = lax.fori_loop(0, _TOPK, ext,
                                 (z16, r[0], jnp.zeros((), jnp.float32)))

        c_gt = jnp.zeros((), jnp.int32)
        for j in range(_TOPK):
            c_gt = c_gt + jnp.sum((r[j] > t8).astype(jnp.int32))
        need_eq = _TOPK - c_gt

        def p3(i, acc):
            v = in_v[pl.ds(i * 16, 16)]
            ge = v >= t8
            out_v[pl.ds(i * 16, 16)] = jnp.where(ge, v, 0.0)
            return acc + ge.astype(jnp.int32)

        acc = lax.fori_loop(0, _NVEC, p3, z16)
        total = jnp.sum(acc)

        @pl.when(total > _TOPK)
        def _fixup():
            def pf(i, run):
                v = in_v[pl.ds(i * 16, 16)]
                eq = v == t8
                eqc = eq.astype(jnp.int32)
                cum = plsc.cumsum(eqc)
                keep = (v > t8) | (eq & ((cum + run) <= need_eq))
                out_v[pl.ds(i * 16, 16)] = jnp.where(keep, v, 0.0)
                return run + jnp.sum(eqc)

            lax.fori_loop(0, _NVEC, pf, jnp.zeros((), jnp.int32))
        pltpu.sync_copy(out_v, out_hbm.at[row, pl.ds(col, _BS)])
        return carry

    lax.fori_loop(0, _UPW, unit_body, 0)


@jax.jit
def kernel(x):
    mesh = plsc.VectorSubcoreMesh(core_axis_name="c", subcore_axis_name="s")
    f = pl.kernel(
        _wta_body,
        out_type=jax.ShapeDtypeStruct((_B, _E), jnp.float32),
        mesh=mesh,
        scratch_types=[
            pltpu.VMEM((_BS,), jnp.float32),
            pltpu.VMEM((_BS,), jnp.float32),
            pltpu.VMEM((_TOPK + 1, 16), jnp.float32),
        ],
        compiler_params=pltpu.CompilerParams(needs_layout_passes=False),
    )
    return f(x)


# speedup vs baseline: 18.8241x; 1.8735x over previous
import jax
import jax.numpy as jnp
from jax import lax
from jax.experimental import pallas as pl
from jax.experimental.pallas import tpu as pltpu
from jax.experimental.pallas import tpu_sc as plsc

_TOPK = 8
_NB = 8
_B = 128
_E = 32768
_BS = _E // _NB
_NVEC = _BS // 16

_info = plsc.get_sparse_core_info()
_NC = _info.num_cores
_NS = _info.num_subcores
_NW = _NC * _NS
_UNITS = _B * _NB
_UPW = _UNITS // _NW


def _unit_compute(in_ref, out_ref, mat_v, lane, neg, z16):
    nch = 4
    span = _NVEC // nch

    def p1(i, r):
        rl = list(r)
        for c in range(nch):
            v = in_ref[pl.ds((c * span + i) * 16, 16)]
            for j in range(_TOPK):
                o = c * _TOPK + j
                hi = jnp.maximum(rl[o], v)
                v = jnp.minimum(rl[o], v)
                rl[o] = hi
        return tuple(rl)

    rr = lax.fori_loop(0, span, p1, (neg,) * (_TOPK * nch))

    def merge2(a, b):
        c = [jnp.maximum(a[j], b[_TOPK - 1 - j]) for j in range(_TOPK)]
        for d in (4, 2, 1):
            for i in range(_TOPK):
                if i & d:
                    continue
                k = i | d
                hi = jnp.maximum(c[i], c[k])
                lo = jnp.minimum(c[i], c[k])
                c[i], c[k] = hi, lo
        return c

    ch = [list(rr[c * _TOPK:(c + 1) * _TOPK]) for c in range(nch)]
    r = merge2(merge2(ch[0], ch[1]), merge2(ch[2], ch[3]))
    for j in range(_TOPK):
        mat_v[j] = r[j]

    def ext(i, c):
        ptr, heads, _ = c
        m = jnp.max(heads)
        f = plsc.all_reduce_ffs(heads == m)
        ptr = ptr + (lane == f).astype(jnp.int32)
        heads = plsc.load_gather(mat_v, [ptr, lane])
        return ptr, heads, m

    _, _, t8 = lax.fori_loop(0, _TOPK, ext,
                             (z16, r[0], jnp.zeros((), jnp.float32)))

    c_gt = jnp.zeros((), jnp.int32)
    for j in range(_TOPK):
        c_gt = c_gt + jnp.sum((r[j] > t8).astype(jnp.int32))
    need_eq = _TOPK - c_gt

    def p3(i, acc):
        for s in range(4):
            v = in_ref[pl.ds((i * 4 + s) * 16, 16)]
            ge = v >= t8
            out_ref[pl.ds((i * 4 + s) * 16, 16)] = jnp.where(ge, v, 0.0)
            acc = acc + ge.astype(jnp.int32)
        return acc

    acc = lax.fori_loop(0, _NVEC // 4, p3, z16)
    total = jnp.sum(acc)

    @pl.when(total > _TOPK)
    def _fixup():
        def pf(i, run):
            v = in_ref[pl.ds(i * 16, 16)]
            eq = v == t8
            eqc = eq.astype(jnp.int32)
            cum = plsc.cumsum(eqc)
            keep = (v > t8) | (eq & ((cum + run) <= need_eq))
            out_ref[pl.ds(i * 16, 16)] = jnp.where(keep, v, 0.0)
            return run + jnp.sum(eqc)

        lax.fori_loop(0, _NVEC, pf, jnp.zeros((), jnp.int32))


def _wta_body(x_hbm, out_hbm, in_v0, in_v1, out_v0, out_v1, mat_v,
              si0, si1, so0, so1):
    wid = lax.axis_index("s") * _NC + lax.axis_index("c")
    lane = lax.iota(jnp.int32, 16)
    neg = jnp.full((16,), -jnp.inf, jnp.float32)
    z16 = jnp.zeros((16,), jnp.int32)
    mat_v[_TOPK] = neg

    in_bufs = (in_v0, in_v1)
    out_bufs = (out_v0, out_v1)
    sins = (si0, si1)
    souts = (so0, so1)
    unit0 = wid * _UPW

    def src_at(unit):
        return x_hbm.at[unit // _NB, pl.ds((unit % _NB) * _BS, _BS)]

    def dst_at(unit):
        return out_hbm.at[unit // _NB, pl.ds((unit % _NB) * _BS, _BS)]

    pltpu.async_copy(src_at(unit0), in_v0, si0)

    def pair_body(h, carry):
        for b in range(2):
            u = 2 * h + b
            unit = unit0 + u

            @pl.when(u + 1 < _UPW)
            def _prefetch():
                pltpu.async_copy(src_at(unit + 1), in_bufs[1 - b],
                                 sins[1 - b])

            pltpu.make_async_copy(src_at(unit), in_bufs[b], sins[b]).wait()

            @pl.when(u >= 2)
            def _drain_out():
                pltpu.make_async_copy(out_bufs[b], dst_at(unit - 2),
                                      souts[b]).wait()

            _unit_compute(in_bufs[b], out_bufs[b], mat_v, lane, neg, z16)
            pltpu.async_copy(out_bufs[b], dst_at(unit), souts[b])
        return carry

    lax.fori_loop(0, _UPW // 2, pair_body, 0)
    pltpu.make_async_copy(out_v0, dst_at(unit0 + _UPW - 2), so0).wait()
    pltpu.make_async_copy(out_v1, dst_at(unit0 + _UPW - 1), so1).wait()


@jax.jit
def kernel(x):
    mesh = plsc.VectorSubcoreMesh(core_axis_name="c", subcore_axis_name="s")
    f = pl.kernel(
        _wta_body,
        out_type=jax.ShapeDtypeStruct((_B, _E), jnp.float32),
        mesh=mesh,
        scratch_types=[
            pltpu.VMEM((_BS,), jnp.float32),
            pltpu.VMEM((_BS,), jnp.float32),
            pltpu.VMEM((_BS,), jnp.float32),
            pltpu.VMEM((_BS,), jnp.float32),
            pltpu.VMEM((_TOPK + 1, 16), jnp.float32),
            pltpu.SemaphoreType.DMA,
            pltpu.SemaphoreType.DMA,
            pltpu.SemaphoreType.DMA,
            pltpu.SemaphoreType.DMA,
        ],
        compiler_params=pltpu.CompilerParams(needs_layout_passes=False),
    )
    return f(x)


# speedup vs baseline: 21.9969x; 1.1686x over previous
import jax
import jax.numpy as jnp
from jax import lax
from jax.experimental import pallas as pl
from jax.experimental.pallas import tpu as pltpu
from jax.experimental.pallas import tpu_sc as plsc

_TOPK = 8
_NB = 8
_B = 128
_E = 32768
_BS = _E // _NB
_NVEC = _BS // 16

_info = plsc.get_sparse_core_info()
_NC = _info.num_cores
_NS = _info.num_subcores
_NW = _NC * _NS
_UNITS = _B * _NB
_UPW = _UNITS // _NW


def _merge2(a, b):
    c = [jnp.maximum(a[j], b[_TOPK - 1 - j]) for j in range(_TOPK)]
    for d in (4, 2, 1):
        for i in range(_TOPK):
            if i & d:
                continue
            k = i | d
            hi = jnp.maximum(c[i], c[k])
            lo = jnp.minimum(c[i], c[k])
            c[i], c[k] = hi, lo
    return c


def _sort8_desc(v):
    v = list(v)
    for k in (2, 4, 8):
        d = k // 2
        while d >= 1:
            for i in range(_TOPK):
                if i & d:
                    continue
                j = i | d
                hi = jnp.maximum(v[i], v[j])
                lo = jnp.minimum(v[i], v[j])
                if (i // k) % 2 == 0:
                    v[i], v[j] = hi, lo
                else:
                    v[i], v[j] = lo, hi
            d //= 2
    return v


def _unit_compute(in_ref, out_ref, mat_v, lane, neg, z16):
    nch = 2
    span = _NVEC // nch
    nbatch = span // _TOPK

    def p1(i, r):
        rl = list(r)
        for c in range(nch):
            base = (c * span + i * _TOPK) * 16
            w = _sort8_desc([in_ref[pl.ds(base + s * 16, 16)]
                             for s in range(_TOPK)])
            rl[c * 8:(c + 1) * 8] = _merge2(rl[c * 8:(c + 1) * 8], w)
        return tuple(rl)

    rr = lax.fori_loop(0, nbatch, p1, (neg,) * (8 * nch))
    r = _merge2(list(rr[0:8]), list(rr[8:16]))

    for s in (1, 2, 4, 8):
        partner = lane ^ s
        for j in range(_TOPK):
            mat_v[j] = r[j]
        b = [plsc.load_gather(mat_v,
                              [jnp.full((16,), j, jnp.int32), partner])
             for j in range(_TOPK)]
        r = _merge2(r, b)

    t8 = r[7]
    c_gt = z16
    for j in range(7):
        c_gt = c_gt + (r[j] > t8).astype(jnp.int32)
    need_eq = _TOPK - c_gt

    def p3(i, acc):
        for s in range(8):
            v = in_ref[pl.ds((i * 8 + s) * 16, 16)]
            ge = v >= t8
            out_ref[pl.ds((i * 8 + s) * 16, 16)] = jnp.where(ge, v, 0.0)
            acc = acc + ge.astype(jnp.int32)
        return acc

    acc = lax.fori_loop(0, _NVEC // 8, p3, z16)
    total = jnp.sum(acc)

    @pl.when(total > _TOPK)
    def _fixup():
        def pf(i, run):
            v = in_ref[pl.ds(i * 16, 16)]
            eq = v == t8
            eqc = eq.astype(jnp.int32)
            cum = plsc.cumsum(eqc)
            keep = (v > t8) | (eq & ((cum + run) <= need_eq))
            out_ref[pl.ds(i * 16, 16)] = jnp.where(keep, v, 0.0)
            return run + jnp.sum(eqc)

        lax.fori_loop(0, _NVEC, pf, z16)


def _wta_body(x_hbm, out_hbm, in_v0, in_v1, out_v0, out_v1, mat_v,
              si0, si1, so0, so1):
    wid = lax.axis_index("s") * _NC + lax.axis_index("c")
    lane = lax.iota(jnp.int32, 16)
    neg = jnp.full((16,), -jnp.inf, jnp.float32)
    z16 = jnp.zeros((16,), jnp.int32)

    in_bufs = (in_v0, in_v1)
    out_bufs = (out_v0, out_v1)
    sins = (si0, si1)
    souts = (so0, so1)
    unit0 = wid * _UPW

    def src_at(unit):
        return x_hbm.at[unit // _NB, pl.ds((unit % _NB) * _BS, _BS)]

    def dst_at(unit):
        return out_hbm.at[unit // _NB, pl.ds((unit % _NB) * _BS, _BS)]

    pltpu.async_copy(src_at(unit0), in_v0, si0)

    def pair_body(h, carry):
        for b in range(2):
            u = 2 * h + b
            unit = unit0 + u

            @pl.when(u + 1 < _UPW)
            def _prefetch():
                pltpu.async_copy(src_at(unit + 1), in_bufs[1 - b],
                                 sins[1 - b])

            pltpu.make_async_copy(src_at(unit), in_bufs[b], sins[b]).wait()

            @pl.when(u >= 2)
            def _drain_out():
                pltpu.make_async_copy(out_bufs[b], dst_at(unit - 2),
                                      souts[b]).wait()

            _unit_compute(in_bufs[b], out_bufs[b], mat_v, lane, neg, z16)
            pltpu.async_copy(out_bufs[b], dst_at(unit), souts[b])
        return carry

    lax.fori_loop(0, _UPW // 2, pair_body, 0)
    pltpu.make_async_copy(out_v0, dst_at(unit0 + _UPW - 2), so0).wait()
    pltpu.make_async_copy(out_v1, dst_at(unit0 + _UPW - 1), so1).wait()


@jax.jit
def kernel(x):
    mesh = plsc.VectorSubcoreMesh(core_axis_name="c", subcore_axis_name="s")
    f = pl.kernel(
        _wta_body,
        out_type=jax.ShapeDtypeStruct((_B, _E), jnp.float32),
        mesh=mesh,
        scratch_types=[
            pltpu.VMEM((_BS,), jnp.float32),
            pltpu.VMEM((_BS,), jnp.float32),
            pltpu.VMEM((_BS,), jnp.float32),
            pltpu.VMEM((_BS,), jnp.float32),
            pltpu.VMEM((_TOPK, 16), jnp.float32),
            pltpu.SemaphoreType.DMA,
            pltpu.SemaphoreType.DMA,
            pltpu.SemaphoreType.DMA,
            pltpu.SemaphoreType.DMA,
        ],
        compiler_params=pltpu.CompilerParams(needs_layout_passes=False),
    )
    return f(x)


# speedup vs baseline: 22.5117x; 1.0234x over previous
import jax
import jax.numpy as jnp
from jax import lax
from jax.experimental import pallas as pl
from jax.experimental.pallas import tpu as pltpu
from jax.experimental.pallas import tpu_sc as plsc

_TOPK = 8
_NB = 8
_B = 128
_E = 32768
_BS = _E // _NB
_NVEC = _BS // 16

_info = plsc.get_sparse_core_info()
_NC = _info.num_cores
_NS = _info.num_subcores
_NW = _NC * _NS
_UNITS = _B * _NB
_UPW = _UNITS // _NW


def _merge2(a, b):
    c = [jnp.maximum(a[j], b[_TOPK - 1 - j]) for j in range(_TOPK)]
    for d in (4, 2, 1):
        for i in range(_TOPK):
            if i & d:
                continue
            k = i | d
            hi = jnp.maximum(c[i], c[k])
            lo = jnp.minimum(c[i], c[k])
            c[i], c[k] = hi, lo
    return c


_OE8 = [(0, 1), (2, 3), (4, 5), (6, 7),
        (0, 2), (1, 3), (4, 6), (5, 7),
        (1, 2), (5, 6),
        (0, 4), (1, 5), (2, 6), (3, 7),
        (2, 4), (3, 5),
        (1, 2), (3, 4), (5, 6)]


def _sort8_desc(v):
    v = list(v)
    for i, j in _OE8:
        hi = jnp.maximum(v[i], v[j])
        lo = jnp.minimum(v[i], v[j])
        v[i], v[j] = hi, lo
    return v


def _unit_compute(in_ref, out_ref, mat_v, lane, neg, z16):
    nch = 2
    span = _NVEC // nch
    nbatch = span // _TOPK

    def p1(i, r):
        rl = list(r)
        for c in range(nch):
            base = (c * span + i * _TOPK) * 16
            w = _sort8_desc([in_ref[pl.ds(base + s * 16, 16)]
                             for s in range(_TOPK)])
            rl[c * 8:(c + 1) * 8] = _merge2(rl[c * 8:(c + 1) * 8], w)
        return tuple(rl)

    rr = lax.fori_loop(0, nbatch, p1, (neg,) * (8 * nch))
    r = _merge2(list(rr[0:8]), list(rr[8:16]))

    for s in (1, 2, 4, 8):
        partner = lane ^ s
        for j in range(_TOPK):
            mat_v[j] = r[j]
        b = [plsc.load_gather(mat_v,
                              [jnp.full((16,), j, jnp.int32), partner])
             for j in range(_TOPK)]
        r = _merge2(r, b)

    t8 = r[7]
    c_gt = z16
    for j in range(7):
        c_gt = c_gt + (r[j] > t8).astype(jnp.int32)
    need_eq = _TOPK - c_gt

    def p3(i, acc):
        for s in range(8):
            v = in_ref[pl.ds((i * 8 + s) * 16, 16)]
            ge = v >= t8
            out_ref[pl.ds((i * 8 + s) * 16, 16)] = jnp.where(ge, v, 0.0)
            acc = acc + ge.astype(jnp.int32)
        return acc

    acc = lax.fori_loop(0, _NVEC // 8, p3, z16)
    total = jnp.sum(acc)

    @pl.when(total > _TOPK)
    def _fixup():
        def pf(i, run):
            v = in_ref[pl.ds(i * 16, 16)]
            eq = v == t8
            eqc = eq.astype(jnp.int32)
            cum = plsc.cumsum(eqc)
            keep = (v > t8) | (eq & ((cum + run) <= need_eq))
            out_ref[pl.ds(i * 16, 16)] = jnp.where(keep, v, 0.0)
            return run + jnp.sum(eqc)

        lax.fori_loop(0, _NVEC, pf, z16)


def _wta_body(x_hbm, out_hbm, in_v0, in_v1, out_v0, out_v1, mat_v,
              si0, si1, so0, so1):
    wid = lax.axis_index("s") * _NC + lax.axis_index("c")
    lane = lax.iota(jnp.int32, 16)
    neg = jnp.full((16,), -jnp.inf, jnp.float32)
    z16 = jnp.zeros((16,), jnp.int32)

    in_bufs = (in_v0, in_v1)
    out_bufs = (out_v0, out_v1)
    sins = (si0, si1)
    souts = (so0, so1)
    unit0 = wid * _UPW

    def src_at(unit):
        return x_hbm.at[unit // _NB, pl.ds((unit % _NB) * _BS, _BS)]

    def dst_at(unit):
        return out_hbm.at[unit // _NB, pl.ds((unit % _NB) * _BS, _BS)]

    pltpu.async_copy(src_at(unit0), in_v0, si0)

    def pair_body(h, carry):
        for b in range(2):
            u = 2 * h + b
            unit = unit0 + u

            @pl.when(u + 1 < _UPW)
            def _prefetch():
                pltpu.async_copy(src_at(unit + 1), in_bufs[1 - b],
                                 sins[1 - b])

            pltpu.make_async_copy(src_at(unit), in_bufs[b], sins[b]).wait()

            @pl.when(u >= 2)
            def _drain_out():
                pltpu.make_async_copy(out_bufs[b], dst_at(unit - 2),
                                      souts[b]).wait()

            _unit_compute(in_bufs[b], out_bufs[b], mat_v, lane, neg, z16)
            pltpu.async_copy(out_bufs[b], dst_at(unit), souts[b])
        return carry

    lax.fori_loop(0, _UPW // 2, pair_body, 0)
    pltpu.make_async_copy(out_v0, dst_at(unit0 + _UPW - 2), so0).wait()
    pltpu.make_async_copy(out_v1, dst_at(unit0 + _UPW - 1), so1).wait()


@jax.jit
def kernel(x):
    mesh = plsc.VectorSubcoreMesh(core_axis_name="c", subcore_axis_name="s")
    f = pl.kernel(
        _wta_body,
        out_type=jax.ShapeDtypeStruct((_B, _E), jnp.float32),
        mesh=mesh,
        scratch_types=[
            pltpu.VMEM((_BS,), jnp.float32),
            pltpu.VMEM((_BS,), jnp.float32),
            pltpu.VMEM((_BS,), jnp.float32),
            pltpu.VMEM((_BS,), jnp.float32),
            pltpu.VMEM((_TOPK, 16), jnp.float32),
            pltpu.SemaphoreType.DMA,
            pltpu.SemaphoreType.DMA,
            pltpu.SemaphoreType.DMA,
            pltpu.SemaphoreType.DMA,
        ],
        compiler_params=pltpu.CompilerParams(needs_layout_passes=False),
    )
    return f(x)
